# Initial kernel scaffold; baseline (speedup 1.0000x reference)
import jax
import jax.numpy as jnp
from jax.experimental import pallas as pl


def kernel(x, W):
    raise NotImplementedError("write your pallas kernel here")


# speedup vs baseline: 1.8764x; 1.8764x over previous
import jax
import jax.numpy as jnp
from jax.experimental import pallas as pl

_E = 64
_K = 2
_TILE = 1024


def _router_kernel(x_ref, w_ref, probs_ref, val_ref, idx_ref):
    x = x_ref[...]
    w = w_ref[...]
    logits = jax.lax.dot_general(
        x, w, (((1,), (1,)), ((), ())), preferred_element_type=jnp.float32
    )
    m = jnp.max(logits, axis=-1, keepdims=True)
    e = jnp.exp(logits - m)
    s = jnp.sum(e, axis=-1, keepdims=True)
    probs = e / s
    probs_ref[...] = probs
    cols = jax.lax.broadcasted_iota(jnp.int32, logits.shape, 1)
    i1 = jnp.argmax(logits, axis=-1)[:, None]
    v1 = jnp.max(probs, axis=-1, keepdims=True)
    masked = jnp.where(cols == i1, -jnp.inf, logits)
    i2 = jnp.argmax(masked, axis=-1)[:, None]
    v2 = jnp.max(jnp.where(cols == i1, -1.0, probs), axis=-1, keepdims=True)
    val_ref[...] = jnp.concatenate([v1, v2], axis=-1)
    idx_ref[...] = jnp.concatenate([i1, i2], axis=-1).astype(jnp.int32)


def kernel(x, W):
    b, n, d = x.shape
    x_flat = x.reshape(b * n, d)
    tokens = b * n
    grid = (tokens // _TILE,)
    probs, vals, idx = pl.pallas_call(
        _router_kernel,
        grid=grid,
        in_specs=[
            pl.BlockSpec((_TILE, d), lambda i: (i, 0)),
            pl.BlockSpec((_E, d), lambda i: (0, 0)),
        ],
        out_specs=[
            pl.BlockSpec((_TILE, _E), lambda i: (i, 0)),
            pl.BlockSpec((_TILE, _K), lambda i: (i, 0)),
            pl.BlockSpec((_TILE, _K), lambda i: (i, 0)),
        ],
        out_shape=[
            jax.ShapeDtypeStruct((tokens, _E), jnp.float32),
            jax.ShapeDtypeStruct((tokens, _K), jnp.float32),
            jax.ShapeDtypeStruct((tokens, _K), jnp.int32),
        ],
    )(x_flat, W)
    return (probs, vals, idx)


# speedup vs baseline: 1.9057x; 1.0156x over previous
import jax
import jax.numpy as jnp
from jax.experimental import pallas as pl
from jax.experimental.pallas import tpu as pltpu

_E = 64
_K = 2
_TILE = 1024


def _router_kernel(x_ref, w_ref, probs_ref, val_ref, idx_ref):
    x = x_ref[...]
    w = w_ref[...]
    logits = jax.lax.dot_general(
        x, w, (((1,), (1,)), ((), ())), preferred_element_type=jnp.float32
    )
    m = jnp.max(logits, axis=-1, keepdims=True)
    i1 = jnp.argmax(logits, axis=-1)[:, None]
    e = jnp.exp(logits - m)
    s = jnp.sum(e, axis=-1, keepdims=True)
    r = 1.0 / s
    probs_ref[...] = e * r
    cols = jax.lax.broadcasted_iota(jnp.int32, logits.shape, 1)
    masked = jnp.where(cols == i1, -jnp.inf, logits)
    m2 = jnp.max(masked, axis=-1, keepdims=True)
    i2 = jnp.argmax(masked, axis=-1)[:, None]
    val_ref[...] = jnp.concatenate([r, jnp.exp(m2 - m) * r], axis=-1)
    idx_ref[...] = jnp.concatenate([i1, i2], axis=-1).astype(jnp.int32)


def kernel(x, W):
    b, n, d = x.shape
    x_flat = x.reshape(b * n, d)
    tokens = b * n
    grid = (tokens // _TILE,)
    probs, vals, idx = pl.pallas_call(
        _router_kernel,
        grid=grid,
        in_specs=[
            pl.BlockSpec((_TILE, d), lambda i: (i, 0)),
            pl.BlockSpec((_E, d), lambda i: (0, 0)),
        ],
        out_specs=[
            pl.BlockSpec((_TILE, _E), lambda i: (i, 0)),
            pl.BlockSpec((_TILE, _K), lambda i: (i, 0)),
            pl.BlockSpec((_TILE, _K), lambda i: (i, 0)),
        ],
        out_shape=[
            jax.ShapeDtypeStruct((tokens, _E), jnp.float32),
            jax.ShapeDtypeStruct((tokens, _K), jnp.float32),
            jax.ShapeDtypeStruct((tokens, _K), jnp.int32),
        ],
        compiler_params=pltpu.CompilerParams(
            dimension_semantics=("parallel",),
        ),
    )(x_flat, W)
    return (probs, vals, idx)


# speedup vs baseline: 2.0678x; 1.0851x over previous
import jax
import jax.numpy as jnp
from jax.experimental import pallas as pl
from jax.experimental.pallas import tpu as pltpu

_E = 64
_K = 2
_TILE = 2048


def _router_kernel(x_ref, w_ref, probs_ref, val_ref, idx_ref):
    x = x_ref[...]
    w = w_ref[...]
    logits = jax.lax.dot_general(
        x, w, (((1,), (1,)), ((), ())), preferred_element_type=jnp.float32
    )
    m = jnp.max(logits, axis=-1, keepdims=True)
    i1 = jnp.argmax(logits, axis=-1)[:, None]
    e = jnp.exp(logits - m)
    s = jnp.sum(e, axis=-1, keepdims=True)
    r = 1.0 / s
    probs_ref[...] = e * r
    cols = jax.lax.broadcasted_iota(jnp.int32, logits.shape, 1)
    masked = jnp.where(cols == i1, -jnp.inf, logits)
    m2 = jnp.max(masked, axis=-1, keepdims=True)
    i2 = jnp.argmax(masked, axis=-1)[:, None]
    val_ref[...] = jnp.concatenate([r, jnp.exp(m2 - m) * r], axis=-1)
    idx_ref[...] = jnp.concatenate([i1, i2], axis=-1).astype(jnp.int32)


def kernel(x, W):
    b, n, d = x.shape
    x_flat = x.reshape(b * n, d)
    tokens = b * n
    grid = (tokens // _TILE,)
    probs, vals, idx = pl.pallas_call(
        _router_kernel,
        grid=grid,
        in_specs=[
            pl.BlockSpec((_TILE, d), lambda i: (i, 0)),
            pl.BlockSpec((_E, d), lambda i: (0, 0)),
        ],
        out_specs=[
            pl.BlockSpec((_TILE, _E), lambda i: (i, 0)),
            pl.BlockSpec((_TILE, _K), lambda i: (i, 0)),
            pl.BlockSpec((_TILE, _K), lambda i: (i, 0)),
        ],
        out_shape=[
            jax.ShapeDtypeStruct((tokens, _E), jnp.float32),
            jax.ShapeDtypeStruct((tokens, _K), jnp.float32),
            jax.ShapeDtypeStruct((tokens, _K), jnp.int32),
        ],
        compiler_params=pltpu.CompilerParams(
            dimension_semantics=("parallel",),
        ),
    )(x_flat, W)
    return (probs, vals, idx)


# speedup vs baseline: 2.1973x; 1.0626x over previous
import jax
import jax.numpy as jnp
from jax.experimental import pallas as pl
from jax.experimental.pallas import tpu as pltpu

_E = 64
_K = 2
_TILE = 4096


def _router_kernel(x_ref, w_ref, probs_ref, val_ref, idx_ref):
    x = x_ref[...]
    w = w_ref[...]
    logits = jax.lax.dot_general(
        x, w, (((1,), (1,)), ((), ())), preferred_element_type=jnp.float32
    )
    m = jnp.max(logits, axis=-1, keepdims=True)
    i1 = jnp.argmax(logits, axis=-1)[:, None]
    e = jnp.exp(logits - m)
    s = jnp.sum(e, axis=-1, keepdims=True)
    r = 1.0 / s
    probs_ref[...] = e * r
    cols = jax.lax.broadcasted_iota(jnp.int32, logits.shape, 1)
    masked = jnp.where(cols == i1, -jnp.inf, logits)
    m2 = jnp.max(masked, axis=-1, keepdims=True)
    i2 = jnp.argmax(masked, axis=-1)[:, None]
    val_ref[...] = jnp.concatenate([r, jnp.exp(m2 - m) * r], axis=-1)
    idx_ref[...] = jnp.concatenate([i1, i2], axis=-1).astype(jnp.int32)


def kernel(x, W):
    b, n, d = x.shape
    x_flat = x.reshape(b * n, d)
    tokens = b * n
    grid = (tokens // _TILE,)
    probs, vals, idx = pl.pallas_call(
        _router_kernel,
        grid=grid,
        in_specs=[
            pl.BlockSpec((_TILE, d), lambda i: (i, 0)),
            pl.BlockSpec((_E, d), lambda i: (0, 0)),
        ],
        out_specs=[
            pl.BlockSpec((_TILE, _E), lambda i: (i, 0)),
            pl.BlockSpec((_TILE, _K), lambda i: (i, 0)),
            pl.BlockSpec((_TILE, _K), lambda i: (i, 0)),
        ],
        out_shape=[
            jax.ShapeDtypeStruct((tokens, _E), jnp.float32),
            jax.ShapeDtypeStruct((tokens, _K), jnp.float32),
            jax.ShapeDtypeStruct((tokens, _K), jnp.int32),
        ],
        compiler_params=pltpu.CompilerParams(
            dimension_semantics=("parallel",),
        ),
    )(x_flat, W)
    return (probs, vals, idx)
